# baseline (speedup 1.0000x reference)
import functools

import jax
import jax.numpy as jnp
from jax import lax
from jax.experimental import pallas as pl
from jax.experimental.pallas import tpu as pltpu
from jax.experimental.pallas import tpu_sc as plsc

Bn, N, K = 2, 10000, 16
D = 128
BN_EPS = 1e-5
NP = Bn * N
NC, NS, L = 2, 16, 16
NW = NC * NS
NB = 10240
PW = NB // NS
C = 8
IDXC = C * K
NCH = PW // C
NJ = D // L
SB = 624
IRF = N * K // 128
IRP = NB * K // 128


def _mm_body(x_ref, w_ref, o_ref):
    o_ref[...] = lax.dot_general(
        x_ref[0].astype(jnp.bfloat16), w_ref[...].astype(jnp.bfloat16),
        (((1,), (1,)), ((), ())), preferred_element_type=jnp.float32)


def _idx_body(knn_ref, o_ref):
    kv = knn_ref[...]
    z = jnp.zeros((IRP - IRF, 128), jnp.int32)
    o_ref[...] = jnp.concatenate(
        [kv[0:IRF], z, kv[IRF:2 * IRF], z], axis=0)


def _bn_body(y_ref, p_ref, w_ref, b_ref, o_ref):
    st = jnp.sum(p_ref[...], axis=0)
    mean = st[0:1, :] * (1.0 / NP)
    msq = st[1:2, :] * (1.0 / NP)
    var = msq - mean * mean
    scale = w_ref[...] * lax.rsqrt(var + BN_EPS)
    shift = b_ref[...] - mean * scale
    o_ref[0] = y_ref[0] * scale + shift


def _sc_body(h_hbm, idx_hbm, y_hbm, part_hbm,
             table_sp, idx_v, stats_v,
             rows_a, rows_b, cent_a, cent_b, out_a, out_b,
             sg_a, sg_b, sc_a, sc_b, so_a, so_b):
    c = lax.axis_index("c")
    s = lax.axis_index("s")
    base = c * NB + s * PW
    lbase = s * PW
    hbase = c * N

    srow = pl.multiple_of(s * SB, 8)
    pltpu.sync_copy(h_hbm.at[pl.ds(hbase + srow, SB)],
                    table_sp.at[pl.ds(srow, SB)])
    pltpu.sync_copy(h_hbm.at[pl.ds(hbase + NS * SB, N - NS * SB)],
                    table_sp.at[pl.ds(NS * SB, N - NS * SB)])

    pltpu.sync_copy(idx_hbm.at[pl.ds(pl.multiple_of(base * K, 8), PW * K)],
                    idx_v)
    for i in range(2 * D // L):
        stats_v[pl.ds(i * L, L)] = jnp.zeros((L,), jnp.float32)
    plsc.subcore_barrier()

    def g_start(g, rows, sem):
        iof = pl.multiple_of(g * IDXC, 8)
        pltpu.async_copy(table_sp.at[idx_v.at[pl.ds(iof, IDXC)]], rows, sem)

    def c_start(g, cent, sem):
        lrow = jnp.minimum(lbase + g * C, N - C)
        pltpu.async_copy(
            h_hbm.at[pl.ds(hbase + pl.multiple_of(lrow, 8), C)], cent, sem)

    def s_start(g, outb, sem):
        pltpu.async_copy(outb, y_hbm.at[pl.ds(base + g * C, C)], sem)

    def in_wait(dst, sem, nrows):
        pltpu.make_async_copy(h_hbm.at[pl.ds(0, nrows)], dst, sem).wait()

    def out_wait(outb, sem):
        pltpu.make_async_copy(outb, y_hbm.at[pl.ds(0, C)], sem).wait()

    def compute(g, rows, cent, outb):
        mvec = jnp.broadcast_to(
            jnp.where(lbase + g * C < N, 1.0, 0.0).astype(jnp.float32), (L,))
        zeros8 = tuple(jnp.zeros((L,), jnp.float32) for _ in range(NJ))

        @pl.loop(0, C, init_carry=(zeros8, zeros8))
        def point_loop(p, carry):
            sums, sqs = carry
            r0 = p * K
            accs = [rows[r0, pl.ds(j * L, L)] for j in range(NJ)]
            for k in range(1, K):
                for j in range(NJ):
                    accs[j] = jnp.maximum(accs[j], rows[r0 + k, pl.ds(j * L, L)])
            new_sums, new_sqs = [], []
            for j in range(NJ):
                yv = accs[j] - cent[p, pl.ds(j * L, L)]
                outb[p, pl.ds(j * L, L)] = yv
                new_sums.append(sums[j] + yv)
                new_sqs.append(sqs[j] + yv * yv)
            return tuple(new_sums), tuple(new_sqs)

        sums, sqs = point_loop
        for j in range(NJ):
            stats_v[pl.ds(j * L, L)] = stats_v[pl.ds(j * L, L)] + sums[j] * mvec
            stats_v[pl.ds(D + j * L, L)] = (stats_v[pl.ds(D + j * L, L)]
                                            + sqs[j] * mvec)

    def do_chunk(g, rows, cent, outb, sg, sc_, so, first, last):
        in_wait(rows, sg, IDXC)
        in_wait(cent, sc_, C)
        if not first:
            out_wait(outb, so)
        compute(g, rows, cent, outb)
        s_start(g, outb, so)
        if not last:
            g_start(g + 2, rows, sg)
            c_start(g + 2, cent, sc_)

    buf_a = (rows_a, cent_a, out_a, sg_a, sc_a, so_a)
    buf_b = (rows_b, cent_b, out_b, sg_b, sc_b, so_b)

    g_start(0, rows_a, sg_a)
    c_start(0, cent_a, sc_a)
    g_start(1, rows_b, sg_b)
    c_start(1, cent_b, sc_b)
    do_chunk(0, *buf_a, first=True, last=False)
    do_chunk(1, *buf_b, first=True, last=False)

    @pl.loop(2, NCH - 2, step=2)
    def chunk_loop(g):
        do_chunk(g, *buf_a, first=False, last=False)
        do_chunk(g + 1, *buf_b, first=False, last=False)

    do_chunk(NCH - 2, *buf_a, first=False, last=True)
    do_chunk(NCH - 1, *buf_b, first=False, last=True)
    out_wait(out_a, so_a)
    out_wait(out_b, so_b)
    pltpu.sync_copy(stats_v, part_hbm.at[c * NS + s])


@functools.partial(
    pl.kernel,
    out_type=(jax.ShapeDtypeStruct((Bn * NB, D), jnp.float32),
              jax.ShapeDtypeStruct((NW, 2 * D), jnp.float32)),
    mesh=plsc.VectorSubcoreMesh(core_axis_name="c", subcore_axis_name="s",
                                num_cores=NC, num_subcores=NS),
    scratch_types=[
        pltpu.VMEM_SHARED((N, D), jnp.float32),
        pltpu.VMEM((PW * K,), jnp.int32),
        pltpu.VMEM((2 * D,), jnp.float32),
        pltpu.VMEM((IDXC, D), jnp.float32),
        pltpu.VMEM((IDXC, D), jnp.float32),
        pltpu.VMEM((C, D), jnp.float32),
        pltpu.VMEM((C, D), jnp.float32),
        pltpu.VMEM((C, D), jnp.float32),
        pltpu.VMEM((C, D), jnp.float32),
        pltpu.SemaphoreType.DMA,
        pltpu.SemaphoreType.DMA,
        pltpu.SemaphoreType.DMA,
        pltpu.SemaphoreType.DMA,
        pltpu.SemaphoreType.DMA,
        pltpu.SemaphoreType.DMA,
    ],
)
def _sc_gather_max(h_hbm, idx_hbm, y_hbm, part_hbm, *scratch):
    _sc_body(h_hbm, idx_hbm, y_hbm, part_hbm, *scratch)


def kernel(x, knn, W, bn_weight, bn_bias):
    h = pl.pallas_call(
        _mm_body,
        grid=(Bn, 5),
        in_specs=[pl.BlockSpec((1, N // 5, D), lambda b, i: (b, i, 0)),
                  pl.BlockSpec((D, D), lambda b, i: (0, 0))],
        out_specs=pl.BlockSpec((N // 5, D), lambda b, i: (b * 5 + i, 0)),
        out_shape=jax.ShapeDtypeStruct((NP, D), jnp.float32),
    )(x, W)

    knn2 = knn.astype(jnp.int32).reshape(Bn * IRF, 128)
    fidx = pl.pallas_call(
        _idx_body,
        grid=(1,),
        in_specs=[pl.BlockSpec((Bn * IRF, 128), lambda i: (0, 0))],
        out_specs=pl.BlockSpec((Bn * IRP, 128), lambda i: (0, 0)),
        out_shape=jax.ShapeDtypeStruct((Bn * IRP, 128), jnp.int32),
    )(knn2).reshape(Bn * NB * K)

    y_flat, partials = _sc_gather_max(h, fidx)
    y3 = y_flat.reshape(Bn, NB, D)

    out = pl.pallas_call(
        _bn_body,
        grid=(Bn, 5),
        in_specs=[pl.BlockSpec((1, N // 5, D), lambda b, i: (b, i, 0)),
                  pl.BlockSpec((NW, 2, D), lambda b, i: (0, 0, 0)),
                  pl.BlockSpec((1, D), lambda b, i: (0, 0)),
                  pl.BlockSpec((1, D), lambda b, i: (0, 0))],
        out_specs=pl.BlockSpec((1, N // 5, D), lambda b, i: (b, i, 0)),
        out_shape=jax.ShapeDtypeStruct((Bn, N, D), jnp.float32),
    )(y3, partials.reshape(NW, 2, D), bn_weight.reshape(1, D),
      bn_bias.reshape(1, D))
    return out


# speedup vs baseline: 1.0215x; 1.0215x over previous
import functools

import jax
import jax.numpy as jnp
from jax import lax
from jax.experimental import pallas as pl
from jax.experimental.pallas import tpu as pltpu
from jax.experimental.pallas import tpu_sc as plsc

Bn, N, K = 2, 10000, 16
D = 128
BN_EPS = 1e-5
NP = Bn * N
NC, NS, L = 2, 16, 16
NW = NC * NS
NB = 10240
PW = NB // NS
C = 8
IDXC = C * K
NCH = PW // C
NJ = D // L
SB = 624
FPAD = (N + NS * PW - PW) * K + PW * K


def _mm_body(x_ref, w_ref, o_ref):
    o_ref[...] = lax.dot_general(
        x_ref[0], w_ref[...], (((1,), (1,)), ((), ())),
        preferred_element_type=jnp.float32)


def _bn_body(y_ref, p_ref, w_ref, b_ref, o_ref):
    st = jnp.sum(p_ref[...], axis=0)
    mean = st[0:1, :] * (1.0 / NP)
    msq = st[1:2, :] * (1.0 / NP)
    var = msq - mean * mean
    scale = w_ref[...] * lax.rsqrt(var + BN_EPS)
    shift = b_ref[...] - mean * scale
    o_ref[0] = y_ref[0] * scale + shift


def _sc_body(h_hbm, idx_hbm, y_hbm, part_hbm,
             table_sp, idx_v, stats_v,
             rows_a, rows_b, cent_a, cent_b, out_a, out_b,
             sg_a, sg_b, sc_a, sc_b, so_a, so_b):
    c = lax.axis_index("c")
    s = lax.axis_index("s")
    base = c * NB + s * PW
    lbase = s * PW
    hbase = c * N

    srow = pl.multiple_of(s * SB, 8)
    pltpu.sync_copy(h_hbm.at[pl.ds(hbase + srow, SB)],
                    table_sp.at[pl.ds(srow, SB)])
    pltpu.sync_copy(h_hbm.at[pl.ds(hbase + NS * SB, N - NS * SB)],
                    table_sp.at[pl.ds(NS * SB, N - NS * SB)])

    ibase = (c * N + s * PW) * K
    pltpu.sync_copy(idx_hbm.at[pl.ds(pl.multiple_of(ibase, 8), PW * K)],
                    idx_v)
    for i in range(2 * D // L):
        stats_v[pl.ds(i * L, L)] = jnp.zeros((L,), jnp.float32)
    plsc.subcore_barrier()

    H2 = IDXC // 2

    def g_start(g, rows, sem):
        iof = pl.multiple_of(g * IDXC, 8)
        pltpu.async_copy(table_sp.at[idx_v.at[pl.ds(iof, H2)]],
                         rows.at[pl.ds(0, H2)], sem)
        pltpu.async_copy(table_sp.at[idx_v.at[pl.ds(iof + H2, H2)]],
                         rows.at[pl.ds(H2, H2)], sem)

    def c_start(g, cent, sem):
        lrow = jnp.minimum(lbase + g * C, N - C)
        pltpu.async_copy(
            h_hbm.at[pl.ds(hbase + pl.multiple_of(lrow, 8), C)], cent, sem)

    def s_start(g, outb, sem):
        pltpu.async_copy(outb, y_hbm.at[pl.ds(base + g * C, C)], sem)

    def in_wait(dst, sem, nrows):
        pltpu.make_async_copy(h_hbm.at[pl.ds(0, nrows)], dst, sem).wait()

    def out_wait(outb, sem):
        pltpu.make_async_copy(outb, y_hbm.at[pl.ds(0, C)], sem).wait()

    def compute(g, rows, cent, outb):
        mvec = jnp.broadcast_to(
            jnp.where(lbase + g * C < N, 1.0, 0.0).astype(jnp.float32), (L,))
        zeros8 = tuple(jnp.zeros((L,), jnp.float32) for _ in range(NJ))

        @pl.loop(0, C, init_carry=(zeros8, zeros8))
        def point_loop(p, carry):
            sums, sqs = carry
            r0 = p * K
            accs = [rows[r0, pl.ds(j * L, L)] for j in range(NJ)]
            for k in range(1, K):
                for j in range(NJ):
                    accs[j] = jnp.maximum(accs[j], rows[r0 + k, pl.ds(j * L, L)])
            new_sums, new_sqs = [], []
            for j in range(NJ):
                yv = accs[j] - cent[p, pl.ds(j * L, L)]
                outb[p, pl.ds(j * L, L)] = yv
                new_sums.append(sums[j] + yv)
                new_sqs.append(sqs[j] + yv * yv)
            return tuple(new_sums), tuple(new_sqs)

        sums, sqs = point_loop
        for j in range(NJ):
            stats_v[pl.ds(j * L, L)] = stats_v[pl.ds(j * L, L)] + sums[j] * mvec
            stats_v[pl.ds(D + j * L, L)] = (stats_v[pl.ds(D + j * L, L)]
                                            + sqs[j] * mvec)

    def do_chunk(g, rows, cent, outb, sg, sc_, so, first, last):
        in_wait(rows, sg, IDXC)
        in_wait(cent, sc_, C)
        if not first:
            out_wait(outb, so)
        compute(g, rows, cent, outb)
        s_start(g, outb, so)
        if not last:
            g_start(g + 2, rows, sg)
            c_start(g + 2, cent, sc_)

    buf_a = (rows_a, cent_a, out_a, sg_a, sc_a, so_a)
    buf_b = (rows_b, cent_b, out_b, sg_b, sc_b, so_b)

    g_start(0, rows_a, sg_a)
    c_start(0, cent_a, sc_a)
    g_start(1, rows_b, sg_b)
    c_start(1, cent_b, sc_b)
    do_chunk(0, *buf_a, first=True, last=False)
    do_chunk(1, *buf_b, first=True, last=False)

    @pl.loop(2, NCH - 2, step=2)
    def chunk_loop(g):
        do_chunk(g, *buf_a, first=False, last=False)
        do_chunk(g + 1, *buf_b, first=False, last=False)

    do_chunk(NCH - 2, *buf_a, first=False, last=True)
    do_chunk(NCH - 1, *buf_b, first=False, last=True)
    out_wait(out_a, so_a)
    out_wait(out_b, so_b)
    pltpu.sync_copy(stats_v, part_hbm.at[c * NS + s])


@functools.partial(
    pl.kernel,
    out_type=(jax.ShapeDtypeStruct((Bn * NB, D), jnp.float32),
              jax.ShapeDtypeStruct((NW, 2 * D), jnp.float32)),
    mesh=plsc.VectorSubcoreMesh(core_axis_name="c", subcore_axis_name="s",
                                num_cores=NC, num_subcores=NS),
    scratch_types=[
        pltpu.VMEM_SHARED((N, D), jnp.float32),
        pltpu.VMEM((PW * K,), jnp.int32),
        pltpu.VMEM((2 * D,), jnp.float32),
        pltpu.VMEM((IDXC, D), jnp.float32),
        pltpu.VMEM((IDXC, D), jnp.float32),
        pltpu.VMEM((C, D), jnp.float32),
        pltpu.VMEM((C, D), jnp.float32),
        pltpu.VMEM((C, D), jnp.float32),
        pltpu.VMEM((C, D), jnp.float32),
        pltpu.SemaphoreType.DMA,
        pltpu.SemaphoreType.DMA,
        pltpu.SemaphoreType.DMA,
        pltpu.SemaphoreType.DMA,
        pltpu.SemaphoreType.DMA,
        pltpu.SemaphoreType.DMA,
    ],
)
def _sc_gather_max(h_hbm, idx_hbm, y_hbm, part_hbm, *scratch):
    _sc_body(h_hbm, idx_hbm, y_hbm, part_hbm, *scratch)


def kernel(x, knn, W, bn_weight, bn_bias):
    h = pl.pallas_call(
        _mm_body,
        grid=(Bn, 5),
        in_specs=[pl.BlockSpec((1, N // 5, D), lambda b, i: (b, i, 0)),
                  pl.BlockSpec((D, D), lambda b, i: (0, 0))],
        out_specs=pl.BlockSpec((N // 5, D), lambda b, i: (b * 5 + i, 0)),
        out_shape=jax.ShapeDtypeStruct((NP, D), jnp.float32),
    )(x, W)

    fidx = jnp.pad(knn.astype(jnp.int32).reshape(NP * K), (0, FPAD - NP * K))

    y_flat, partials = _sc_gather_max(h, fidx)
    y3 = y_flat.reshape(Bn, NB, D)

    out = pl.pallas_call(
        _bn_body,
        grid=(Bn, 5),
        in_specs=[pl.BlockSpec((1, N // 5, D), lambda b, i: (b, i, 0)),
                  pl.BlockSpec((NW, 2, D), lambda b, i: (0, 0, 0)),
                  pl.BlockSpec((1, D), lambda b, i: (0, 0)),
                  pl.BlockSpec((1, D), lambda b, i: (0, 0))],
        out_specs=pl.BlockSpec((1, N // 5, D), lambda b, i: (b, i, 0)),
        out_shape=jax.ShapeDtypeStruct((Bn, N, D), jnp.float32),
    )(y3, partials.reshape(NW, 2, D), bn_weight.reshape(1, D),
      bn_bias.reshape(1, D))
    return out
